# baseline (speedup 1.0000x reference)
import jax
import jax.numpy as jnp
from jax.experimental import pallas as pl
from jax.experimental.pallas import tpu as pltpu

BANK = 1_000_000
EMB = 32
BS = 16384
K = 8
RK = BANK // K
C = 512
NC = BS // C


def _body(ptr_ref, emb_ref, q_ref, out_ref, bulk_sem, win_sem):
    for k in range(K):
        pltpu.make_async_copy(
            q_ref.at[pl.ds(k * RK, RK), :],
            out_ref.at[pl.ds(k * RK, RK), :],
            bulk_sem.at[k],
        ).start()
    for k in range(K):
        pltpu.make_async_copy(
            q_ref.at[pl.ds(k * RK, RK), :],
            out_ref.at[pl.ds(k * RK, RK), :],
            bulk_sem.at[k],
        ).wait()

    p = ptr_ref[0]
    for c in range(NC):
        off = jax.lax.rem(p + c * C, BANK)
        whole = off <= BANK - C

        @pl.when(whole)
        def _():
            pltpu.make_async_copy(
                emb_ref.at[pl.ds(c * C, C), :],
                out_ref.at[pl.ds(off, C), :],
                win_sem.at[c],
            ).start()

        @pl.when(jnp.logical_not(whole))
        def _():
            def row(r, _):
                d = jax.lax.rem(off + r, BANK)
                cp = pltpu.make_async_copy(
                    emb_ref.at[pl.ds(c * C + r, 1), :],
                    out_ref.at[pl.ds(d, 1), :],
                    win_sem.at[c],
                )
                cp.start()
                cp.wait()
                return 0

            jax.lax.fori_loop(0, C, row, 0)

    for c in range(NC):
        off = jax.lax.rem(p + c * C, BANK)

        @pl.when(off <= BANK - C)
        def _():
            pltpu.make_async_copy(
                emb_ref.at[pl.ds(c * C, C), :],
                out_ref.at[pl.ds(off, C), :],
                win_sem.at[c],
            ).wait()


def kernel(embeddings, queue, ptr):
    p = jax.lax.rem(jnp.asarray(ptr, jnp.int32), BANK)
    return pl.pallas_call(
        _body,
        in_specs=[
            pl.BlockSpec(memory_space=pltpu.SMEM),
            pl.BlockSpec(memory_space=pl.ANY),
            pl.BlockSpec(memory_space=pl.ANY),
        ],
        out_specs=pl.BlockSpec(memory_space=pl.ANY),
        out_shape=jax.ShapeDtypeStruct((BANK, EMB), jnp.float32),
        scratch_shapes=[
            pltpu.SemaphoreType.DMA((K,)),
            pltpu.SemaphoreType.DMA((NC,)),
        ],
    )(p.reshape(1), embeddings, queue)


# speedup vs baseline: 17.3431x; 17.3431x over previous
import jax
import jax.numpy as jnp
from jax.experimental import pallas as pl
from jax.experimental.pallas import tpu as pltpu

BANK = 1_000_000
EMB = 32
BS = 16384
BR = 4_000
GRID = BANK // BR
EPAD = BS + 2 * BR


def _body(ptr_ref, emb_ref, q_ref, out_ref):
    i = pl.program_id(0)
    s = i * BR
    p = ptr_ref[0]

    o = jax.lax.rem(s - p + BANK, BANK)
    has_window = jnp.logical_or(o < BS, o > BANK - BR)

    @pl.when(has_window)
    def _():
        b = jnp.where(o >= BANK - BR, o - BANK, o)
        b = jnp.clip(b, -BR, BS)
        emb_slice = emb_ref[pl.ds(b + BR, BR), :]

        j = jax.lax.broadcasted_iota(jnp.int32, (BR, 1), 0)
        d0 = o + j
        delta = jnp.where(d0 >= BANK, d0 - BANK, d0)
        take = delta < BS
        out_ref[:, :] = jnp.where(take, emb_slice, q_ref[:, :])

    @pl.when(jnp.logical_not(has_window))
    def _():
        out_ref[:, :] = q_ref[:, :]


def kernel(embeddings, queue, ptr):
    emb_p = jnp.pad(embeddings, ((BR, BR), (0, 0)))
    p = jax.lax.rem(jnp.asarray(ptr, jnp.int32), BANK)
    return pl.pallas_call(
        _body,
        grid=(GRID,),
        in_specs=[
            pl.BlockSpec(memory_space=pltpu.SMEM),
            pl.BlockSpec((EPAD, EMB), lambda i: (0, 0)),
            pl.BlockSpec((BR, EMB), lambda i: (i, 0)),
        ],
        out_specs=pl.BlockSpec((BR, EMB), lambda i: (i, 0)),
        out_shape=jax.ShapeDtypeStruct((BANK, EMB), jnp.float32),
    )(p.reshape(1), emb_p, queue)
